# Initial kernel scaffold; baseline (speedup 1.0000x reference)
import jax
import jax.numpy as jnp
from jax.experimental import pallas as pl


def kernel(x, node_map, edge_index, softgate, W1, b1, W2, b2):
    raise NotImplementedError("write your pallas kernel here")


# speedup vs baseline: 6.2720x; 6.2720x over previous
import jax
import jax.numpy as jnp
from jax import lax
from jax.experimental import pallas as pl
from jax.experimental.pallas import tpu as pltpu
from jax.experimental.pallas import tpu_sc as plsc

N = 10000
D = 128
NEXP = 8
E = 320000

NPAD = 10240
TRASH = N
E_PAD = 323584
B = 128
NB_AGG = E_PAD // 16 // B
NB_DEG = E_PAD // 32 // B
STRIPE = NPAD // 16
CW = 64
NCH = 2 * NEXP
BN = 256
NBLK = NPAD // BN

_F32 = jnp.float32
_I32 = jnp.int32


def _mesh():
    return plsc.VectorSubcoreMesh(
        core_axis_name="c", subcore_axis_name="s", num_cores=2, num_subcores=16
    )


def _sc_prep_body(x_hbm, nmap_hbm, dst_hbm, xs_out, degp_out,
                  nidx_v, didx_v, ones_v, rows_v, zb_v, dacc_sh, sem):
    c = lax.axis_index("c")
    s = lax.axis_index("s")
    wid = s * 2 + c
    lane = lax.iota(_I32, 16)
    e0 = jnp.where(lane == 0, 1.0, 0.0).astype(_F32)
    zv = jnp.zeros((16,), _F32)

    @pl.loop(0, B)
    def _init_ones(i):
        for j in range(CW // 16):
            ones_v[i, pl.ds(j * 16, 16)] = e0 if j == 0 else zv

    @pl.loop(0, 128)
    def _init_zb(i):
        for j in range(CW // 16):
            zb_v[i, pl.ds(j * 16, 16)] = zv

    for z in range(5):
        pltpu.sync_copy(zb_v, dacc_sh.at[pl.ds(s * STRIPE + z * 128, 128)])
    plsc.subcore_barrier()

    dbase = wid * (E_PAD // 32)

    @pl.loop(0, NB_DEG)
    def _deg(b):
        off = pl.multiple_of(dbase + b * B, B)
        pltpu.sync_copy(dst_hbm.at[pl.ds(off, B)], didx_v)
        pltpu.sync_copy(ones_v, dacc_sh.at[didx_v], add=True)

    nbase = wid * (NPAD // 32)
    for q in range(5):
        qoff = pl.multiple_of(nbase + q * 64, 8)
        pltpu.sync_copy(nmap_hbm.at[pl.ds(qoff, 64)], nidx_v)
        pltpu.async_copy(x_hbm.at[nidx_v], rows_v, sem).wait()
        pltpu.sync_copy(rows_v, xs_out.at[pl.ds(qoff, 64)])

    plsc.subcore_barrier()
    pltpu.sync_copy(
        dacc_sh.at[pl.ds(s * STRIPE, STRIPE)],
        degp_out.at[c].at[pl.ds(s * STRIPE, STRIPE)],
    )


def _sc_prep(x, nmap_pad, dst_pad):
    f = pl.kernel(
        _sc_prep_body,
        out_type=(
            jax.ShapeDtypeStruct((NPAD, D), _F32),
            jax.ShapeDtypeStruct((2, NPAD, CW), _F32),
        ),
        mesh=_mesh(),
        scratch_types=[
            pltpu.VMEM((64,), _I32),
            pltpu.VMEM((B,), _I32),
            pltpu.VMEM((B, CW), _F32),
            pltpu.VMEM((64, D), _F32),
            pltpu.VMEM((128, CW), _F32),
            pltpu.VMEM_SHARED((NPAD, CW), _F32),
            pltpu.SemaphoreType.DMA,
        ],
        compiler_params=pltpu.CompilerParams(use_tc_tiling_on_sc=False),
    )
    return f(x, nmap_pad, dst_pad)


def _sc_agg_body(h_hbm, src_hbm3, dst_hbm3, s_out,
                 srcv, dstv, rows_v, zb_v, acc_sh, sem):
    c = lax.axis_index("c")
    s = lax.axis_index("s")
    zv = jnp.zeros((16,), _F32)

    @pl.loop(0, 128)
    def _init_zb(i):
        for j in range(CW // 16):
            zb_v[i, pl.ds(j * 16, 16)] = zv

    pltpu.sync_copy(src_hbm3.at[s], srcv)
    pltpu.sync_copy(dst_hbm3.at[s], dstv)

    for q in range(NCH):
        @pl.when(c == (q % NEXP) // 4)
        def _chunk():
            for z in range(5):
                pltpu.sync_copy(zb_v, acc_sh.at[pl.ds(s * STRIPE + z * 128, 128)])
            plsc.subcore_barrier()

            @pl.loop(0, NB_AGG)
            def _edges(b):
                pltpu.async_copy(h_hbm.at[q].at[srcv.at[b]], rows_v, sem).wait()
                pltpu.sync_copy(rows_v, acc_sh.at[dstv.at[b]], add=True)

            plsc.subcore_barrier()
            pltpu.sync_copy(
                acc_sh.at[pl.ds(s * STRIPE, STRIPE)],
                s_out.at[q].at[pl.ds(s * STRIPE, STRIPE)],
            )


def _sc_agg(h, src3, dst3):
    f = pl.kernel(
        _sc_agg_body,
        out_type=jax.ShapeDtypeStruct((NCH, NPAD, CW), _F32),
        mesh=_mesh(),
        scratch_types=[
            pltpu.VMEM((NB_AGG, B), _I32),
            pltpu.VMEM((NB_AGG, B), _I32),
            pltpu.VMEM((B, CW), _F32),
            pltpu.VMEM((128, CW), _F32),
            pltpu.VMEM_SHARED((NPAD, CW), _F32),
            pltpu.SemaphoreType.DMA,
        ],
        compiler_params=pltpu.CompilerParams(use_tc_tiling_on_sc=False),
    )
    return f(h, src3, dst3)


def _tc1_body(xs_ref, dga_ref, dgb_ref, sg_ref, w1_ref, h1_ref, g_ref):
    xs = xs_ref[...]
    logits = lax.dot_general(xs, sg_ref[...], (((1,), (1,)), ((), ())),
                             preferred_element_type=_F32)
    m = jnp.max(logits, axis=1, keepdims=True)
    p = jnp.exp(logits - m)
    g = p / jnp.sum(p, axis=1, keepdims=True)
    g_ref[...] = g
    dis = lax.rsqrt(dga_ref[...] + dgb_ref[...] + 1.0)
    for e in range(NEXP):
        he = lax.dot_general(g[:, e:e + 1] * xs, w1_ref[e],
                             (((1,), (0,)), ((), ())),
                             preferred_element_type=_F32)
        he = dis * he
        h1_ref[e] = he[:, :CW]
        h1_ref[NEXP + e] = he[:, CW:]


def _tc1(xs, dga, dgb, softgate, w1):
    return pl.pallas_call(
        _tc1_body,
        grid=(NBLK,),
        in_specs=[
            pl.BlockSpec((BN, D), lambda i: (i, 0)),
            pl.BlockSpec((BN, 1), lambda i: (i, 0)),
            pl.BlockSpec((BN, 1), lambda i: (i, 0)),
            pl.BlockSpec((NEXP, D), lambda i: (0, 0)),
            pl.BlockSpec((NEXP, D, D), lambda i: (0, 0, 0)),
        ],
        out_specs=[
            pl.BlockSpec((NCH, BN, CW), lambda i: (0, i, 0)),
            pl.BlockSpec((BN, NEXP), lambda i: (i, 0)),
        ],
        out_shape=[
            jax.ShapeDtypeStruct((NCH, NPAD, CW), _F32),
            jax.ShapeDtypeStruct((NPAD, NEXP), _F32),
        ],
    )(xs, dga, dgb, softgate, w1)


def _tc2_body(s1_ref, h1_ref, dga_ref, dgb_ref, b1_ref, w2_ref, h2_ref):
    dis = lax.rsqrt(dga_ref[...] + dgb_ref[...] + 1.0)
    for e in range(NEXP):
        u = jnp.concatenate(
            [s1_ref[e] + h1_ref[e], s1_ref[NEXP + e] + h1_ref[NEXP + e]], axis=1)
        z = jnp.maximum(dis * u + b1_ref[e], 0.0)
        w = dis * lax.dot_general(z, w2_ref[e], (((1,), (0,)), ((), ())),
                                  preferred_element_type=_F32)
        h2_ref[e] = w[:, :CW]
        h2_ref[NEXP + e] = w[:, CW:]


def _tc2(s1, h1, dga, dgb, b1, w2):
    return pl.pallas_call(
        _tc2_body,
        grid=(NBLK,),
        in_specs=[
            pl.BlockSpec((NCH, BN, CW), lambda i: (0, i, 0)),
            pl.BlockSpec((NCH, BN, CW), lambda i: (0, i, 0)),
            pl.BlockSpec((BN, 1), lambda i: (i, 0)),
            pl.BlockSpec((BN, 1), lambda i: (i, 0)),
            pl.BlockSpec((NEXP, D), lambda i: (0, 0)),
            pl.BlockSpec((NEXP, D, D), lambda i: (0, 0, 0)),
        ],
        out_specs=pl.BlockSpec((NCH, BN, CW), lambda i: (0, i, 0)),
        out_shape=jax.ShapeDtypeStruct((NCH, NPAD, CW), _F32),
    )(s1, h1, dga, dgb, b1, w2)


def _tc3_body(s2_ref, h2_ref, dga_ref, dgb_ref, b2_ref, g_ref, out_ref):
    dis = lax.rsqrt(dga_ref[...] + dgb_ref[...] + 1.0)
    g = g_ref[...]
    acc = jnp.zeros((BN, D), _F32)
    for e in range(NEXP):
        u = jnp.concatenate(
            [s2_ref[e] + h2_ref[e], s2_ref[NEXP + e] + h2_ref[NEXP + e]], axis=1)
        acc = acc + g[:, e:e + 1] * (dis * u + b2_ref[e])
    out_ref[...] = acc


def _tc3(s2, h2, dga, dgb, b2, g):
    return pl.pallas_call(
        _tc3_body,
        grid=(NBLK,),
        in_specs=[
            pl.BlockSpec((NCH, BN, CW), lambda i: (0, i, 0)),
            pl.BlockSpec((NCH, BN, CW), lambda i: (0, i, 0)),
            pl.BlockSpec((BN, 1), lambda i: (i, 0)),
            pl.BlockSpec((BN, 1), lambda i: (i, 0)),
            pl.BlockSpec((NEXP, D), lambda i: (0, 0)),
            pl.BlockSpec((BN, NEXP), lambda i: (i, 0)),
        ],
        out_specs=pl.BlockSpec((BN, D), lambda i: (i, 0)),
        out_shape=jax.ShapeDtypeStruct((NPAD, D), _F32),
    )(s2, h2, dga, dgb, b2, g)


def kernel(x, node_map, edge_index, softgate, W1, b1, W2, b2):
    nm = node_map.astype(_I32)
    src = edge_index[0].astype(_I32)
    dst = edge_index[1].astype(_I32)
    nm_pad = jnp.concatenate([nm, jnp.zeros((NPAD - N,), _I32)])
    src_pad = jnp.concatenate([src, jnp.zeros((E_PAD - E,), _I32)])
    dst_pad = jnp.concatenate([dst, jnp.full((E_PAD - E,), TRASH, _I32)])
    src3 = src_pad.reshape(16, NB_AGG, B)
    dst3 = dst_pad.reshape(16, NB_AGG, B)

    xs_pad, degp = _sc_prep(x, nm_pad, dst_pad)
    dga = degp[0, :, 0:1]
    dgb = degp[1, :, 0:1]

    h1, g = _tc1(xs_pad, dga, dgb, softgate, W1)
    s1 = _sc_agg(h1, src3, dst3)
    h2 = _tc2(s1, h1, dga, dgb, b1, W2)
    s2 = _sc_agg(h2, src3, dst3)
    out_full = _tc3(s2, h2, dga, dgb, b2, g)
    return out_full[:N]


# speedup vs baseline: 10.8651x; 1.7323x over previous
import jax
import jax.numpy as jnp
from jax import lax
from jax.experimental import pallas as pl
from jax.experimental.pallas import tpu as pltpu
from jax.experimental.pallas import tpu_sc as plsc

N = 10000
D = 128
NEXP = 8
E = 320000

NPAD = 10240
TRASH = N
E_PAD = 327680
B = 128
NB_DEG = E_PAD // 32 // B
BOP = 128
NBOP = E_PAD // 16 // BOP
STRIPE = NPAD // 16
CW = 128
NCH = NEXP
BN = 256
NBLK = NPAD // BN

_F32 = jnp.float32
_BF16 = jnp.bfloat16
_I32 = jnp.int32


def _mesh():
    return plsc.VectorSubcoreMesh(
        core_axis_name="c", subcore_axis_name="s", num_cores=2, num_subcores=16
    )


def _sc_prep_body(x_hbm, nmap_hbm, dst_hbm, xs_out, degp_out,
                  nidx_v, didx_v, ones_v, rows_v, zb_v, dacc_sh, sem):
    c = lax.axis_index("c")
    s = lax.axis_index("s")
    wid = s * 2 + c
    lane = lax.iota(_I32, 16)
    e0 = jnp.where(lane == 0, 1.0, 0.0).astype(_F32)
    zv = jnp.zeros((16,), _F32)

    @pl.loop(0, B)
    def _init_ones(i):
        ones_v[i, :] = e0

    @pl.loop(0, 128)
    def _init_zb(i):
        zb_v[i, :] = zv

    for z in range(5):
        pltpu.sync_copy(zb_v, dacc_sh.at[pl.ds(s * STRIPE + z * 128, 128)])
    plsc.subcore_barrier()

    dbase = wid * (E_PAD // 32)

    @pl.loop(0, NB_DEG)
    def _deg(b):
        off = pl.multiple_of(dbase + b * B, B)
        pltpu.sync_copy(dst_hbm.at[pl.ds(off, B)], didx_v)
        pltpu.sync_copy(ones_v, dacc_sh.at[didx_v], add=True)

    nbase = wid * (NPAD // 32)
    for q in range(5):
        qoff = pl.multiple_of(nbase + q * 64, 8)
        pltpu.sync_copy(nmap_hbm.at[pl.ds(qoff, 64)], nidx_v)
        pltpu.async_copy(x_hbm.at[nidx_v], rows_v, sem).wait()
        pltpu.sync_copy(rows_v, xs_out.at[pl.ds(qoff, 64)])

    plsc.subcore_barrier()
    pltpu.sync_copy(
        dacc_sh.at[pl.ds(s * STRIPE, STRIPE)],
        degp_out.at[c].at[pl.ds(s * STRIPE, STRIPE)],
    )


def _sc_prep(x, nmap_pad, dst_pad):
    f = pl.kernel(
        _sc_prep_body,
        out_type=(
            jax.ShapeDtypeStruct((NPAD, D), _F32),
            jax.ShapeDtypeStruct((2, NPAD, 16), _F32),
        ),
        mesh=_mesh(),
        scratch_types=[
            pltpu.VMEM((64,), _I32),
            pltpu.VMEM((B,), _I32),
            pltpu.VMEM((B, 16), _F32),
            pltpu.VMEM((64, D), _F32),
            pltpu.VMEM((128, 16), _F32),
            pltpu.VMEM_SHARED((NPAD, 16), _F32),
            pltpu.SemaphoreType.DMA,
        ],
        compiler_params=pltpu.CompilerParams(use_tc_tiling_on_sc=False),
    )
    return f(x, nmap_pad, dst_pad)


def _sc_agg_body(h_hbm, src_hbm3, dst_hbm3, dummy_hbm, s_out,
                 srcv, dstv, rows0, rows1, rows2, rows3, zb_v, acc_sh,
                 gsem0, gsem1, gsem2, gsem3, ssem0, ssem1, ssem2, ssem3):
    c = lax.axis_index("c")
    s = lax.axis_index("s")
    zv = jnp.zeros((32,), _BF16)
    rows = (rows0, rows1, rows2, rows3)
    gsem = (gsem0, gsem1, gsem2, gsem3)
    ssem = (ssem0, ssem1, ssem2, ssem3)

    @pl.loop(0, 64)
    def _init_zb(i):
        for j in range(CW // 32):
            zb_v[i, pl.ds(j * 32, 32)] = zv

    pltpu.sync_copy(src_hbm3.at[s], srcv)
    pltpu.sync_copy(dst_hbm3.at[s], dstv)

    for q in range(NCH):
        def _fire_g(st, b, _q=q):
            pltpu.async_copy(h_hbm.at[_q].at[srcv.at[b]], rows[st], gsem[st])

        def _fire_s(st, b):
            pltpu.async_copy(rows[st], acc_sh.at[dstv.at[b]], ssem[st],
                             add=True)

        def _drain_g(st):
            pltpu.make_async_copy(dummy_hbm, rows[st], gsem[st]).wait()

        def _drain_s(st):
            pltpu.make_async_copy(rows[st], acc_sh.at[pl.ds(0, BOP)],
                                  ssem[st]).wait()

        @pl.when(c == q // 4)
        def _chunk():
            for z in range(10):
                pltpu.sync_copy(zb_v, acc_sh.at[pl.ds(s * STRIPE + z * 64, 64)])
            plsc.subcore_barrier()

            for st in range(4):
                _fire_g(st, st)

            @pl.loop(0, (NBOP - 4) // 4)
            def _quads(t):
                b = 4 * t
                for st in range(4):
                    _drain_g(st)
                    _fire_s(st, b + st)
                for st in range(4):
                    _drain_s(st)
                    _fire_g(st, b + 4 + st)

            for st in range(4):
                _drain_g(st)
                _fire_s(st, NBOP - 4 + st)
            for st in range(4):
                _drain_s(st)

            plsc.subcore_barrier()
            pltpu.sync_copy(
                acc_sh.at[pl.ds(s * STRIPE, STRIPE)],
                s_out.at[q].at[pl.ds(s * STRIPE, STRIPE)],
            )


_SC_AGG_SINGLETON = None


def _sc_agg(h, src3, dst3):
    global _SC_AGG_SINGLETON
    if _SC_AGG_SINGLETON is not None:
        return _SC_AGG_SINGLETON(h, src3, dst3, jnp.zeros((BOP, CW), _BF16))
    f = pl.kernel(
        _sc_agg_body,
        out_type=jax.ShapeDtypeStruct((NCH, NPAD, CW), _BF16),
        mesh=_mesh(),
        scratch_types=[
            pltpu.VMEM((NBOP, BOP), _I32),
            pltpu.VMEM((NBOP, BOP), _I32),
            pltpu.VMEM((BOP, CW), _BF16),
            pltpu.VMEM((BOP, CW), _BF16),
            pltpu.VMEM((BOP, CW), _BF16),
            pltpu.VMEM((BOP, CW), _BF16),
            pltpu.VMEM((64, CW), _BF16),
            pltpu.VMEM_SHARED((NPAD, CW), _BF16),
            pltpu.SemaphoreType.DMA,
            pltpu.SemaphoreType.DMA,
            pltpu.SemaphoreType.DMA,
            pltpu.SemaphoreType.DMA,
            pltpu.SemaphoreType.DMA,
            pltpu.SemaphoreType.DMA,
            pltpu.SemaphoreType.DMA,
            pltpu.SemaphoreType.DMA,
        ],
        compiler_params=pltpu.CompilerParams(use_tc_tiling_on_sc=False),
    )
    _SC_AGG_SINGLETON = f
    return f(h, src3, dst3, jnp.zeros((BOP, CW), _BF16))


def _tc1_body(xs_ref, dga_ref, dgb_ref, sg_ref, w1_ref, h1_ref, g_ref):
    xs = xs_ref[...]
    logits = lax.dot_general(xs, sg_ref[...], (((1,), (1,)), ((), ())),
                             preferred_element_type=_F32)
    m = jnp.max(logits, axis=1, keepdims=True)
    p = jnp.exp(logits - m)
    g = p / jnp.sum(p, axis=1, keepdims=True)
    g_ref[...] = g
    dis = lax.rsqrt(dga_ref[...] + dgb_ref[...] + 1.0)
    for e in range(NEXP):
        he = lax.dot_general(g[:, e:e + 1] * xs, w1_ref[e],
                             (((1,), (0,)), ((), ())),
                             preferred_element_type=_F32)
        h1_ref[e] = (dis * he).astype(_BF16)


def _tc1(xs, dga, dgb, softgate, w1):
    return pl.pallas_call(
        _tc1_body,
        grid=(NBLK,),
        in_specs=[
            pl.BlockSpec((BN, D), lambda i: (i, 0)),
            pl.BlockSpec((BN, 1), lambda i: (i, 0)),
            pl.BlockSpec((BN, 1), lambda i: (i, 0)),
            pl.BlockSpec((NEXP, D), lambda i: (0, 0)),
            pl.BlockSpec((NEXP, D, D), lambda i: (0, 0, 0)),
        ],
        out_specs=[
            pl.BlockSpec((NCH, BN, CW), lambda i: (0, i, 0)),
            pl.BlockSpec((BN, NEXP), lambda i: (i, 0)),
        ],
        out_shape=[
            jax.ShapeDtypeStruct((NCH, NPAD, CW), _BF16),
            jax.ShapeDtypeStruct((NPAD, NEXP), _F32),
        ],
    )(xs, dga, dgb, softgate, w1)


def _tc2_body(s1_ref, h1_ref, dga_ref, dgb_ref, b1_ref, w2_ref, h2_ref):
    dis = lax.rsqrt(dga_ref[...] + dgb_ref[...] + 1.0)
    for e in range(NEXP):
        u = s1_ref[e].astype(_F32) + h1_ref[e].astype(_F32)
        z = jnp.maximum(dis * u + b1_ref[e], 0.0)
        w = dis * lax.dot_general(z, w2_ref[e], (((1,), (0,)), ((), ())),
                                  preferred_element_type=_F32)
        h2_ref[e] = w.astype(_BF16)


def _tc2(s1, h1, dga, dgb, b1, w2):
    return pl.pallas_call(
        _tc2_body,
        grid=(NBLK,),
        in_specs=[
            pl.BlockSpec((NCH, BN, CW), lambda i: (0, i, 0)),
            pl.BlockSpec((NCH, BN, CW), lambda i: (0, i, 0)),
            pl.BlockSpec((BN, 1), lambda i: (i, 0)),
            pl.BlockSpec((BN, 1), lambda i: (i, 0)),
            pl.BlockSpec((NEXP, D), lambda i: (0, 0)),
            pl.BlockSpec((NEXP, D, D), lambda i: (0, 0, 0)),
        ],
        out_specs=pl.BlockSpec((NCH, BN, CW), lambda i: (0, i, 0)),
        out_shape=jax.ShapeDtypeStruct((NCH, NPAD, CW), _BF16),
    )(s1, h1, dga, dgb, b1, w2)


def _tc3_body(s2_ref, h2_ref, dga_ref, dgb_ref, b2_ref, g_ref, out_ref):
    dis = lax.rsqrt(dga_ref[...] + dgb_ref[...] + 1.0)
    g = g_ref[...]
    acc = jnp.zeros((BN, D), _F32)
    for e in range(NEXP):
        u = s2_ref[e].astype(_F32) + h2_ref[e].astype(_F32)
        acc = acc + g[:, e:e + 1] * (dis * u + b2_ref[e])
    out_ref[...] = acc


def _tc3(s2, h2, dga, dgb, b2, g):
    return pl.pallas_call(
        _tc3_body,
        grid=(NBLK,),
        in_specs=[
            pl.BlockSpec((NCH, BN, CW), lambda i: (0, i, 0)),
            pl.BlockSpec((NCH, BN, CW), lambda i: (0, i, 0)),
            pl.BlockSpec((BN, 1), lambda i: (i, 0)),
            pl.BlockSpec((BN, 1), lambda i: (i, 0)),
            pl.BlockSpec((NEXP, D), lambda i: (0, 0)),
            pl.BlockSpec((BN, NEXP), lambda i: (i, 0)),
        ],
        out_specs=pl.BlockSpec((BN, D), lambda i: (i, 0)),
        out_shape=jax.ShapeDtypeStruct((NPAD, D), _F32),
    )(s2, h2, dga, dgb, b2, g)


def kernel(x, node_map, edge_index, softgate, W1, b1, W2, b2):
    nm = node_map.astype(_I32)
    src = edge_index[0].astype(_I32)
    dst = edge_index[1].astype(_I32)
    nm_pad = jnp.concatenate([nm, jnp.zeros((NPAD - N,), _I32)])
    src_pad = jnp.concatenate([src, jnp.zeros((E_PAD - E,), _I32)])
    dst_pad = jnp.concatenate([dst, jnp.full((E_PAD - E,), TRASH, _I32)])
    src3 = src_pad.reshape(16, NBOP, BOP)
    dst3 = dst_pad.reshape(16, NBOP, BOP)

    xs_pad, degp = _sc_prep(x, nm_pad, dst_pad)
    dga = degp[0, :, 0:1]
    dgb = degp[1, :, 0:1]

    h1, g = _tc1(xs_pad, dga, dgb, softgate, W1)
    s1 = _sc_agg(h1, src3, dst3)
    h2 = _tc2(s1, h1, dga, dgb, b1, W2)
    s2 = _sc_agg(h2, src3, dst3)
    out_full = _tc3(s2, h2, dga, dgb, b2, g)
    return out_full[:N]


# speedup vs baseline: 10.9993x; 1.0124x over previous
import jax
import jax.numpy as jnp
from jax import lax
from jax.experimental import pallas as pl
from jax.experimental.pallas import tpu as pltpu
from jax.experimental.pallas import tpu_sc as plsc

N = 10000
D = 128
NEXP = 8
E = 320000

NPAD = 10240
TRASH = N
E_PAD = 327680
B = 128
NB_DEG = E_PAD // 32 // B
BOP = 128
NBOP = E_PAD // 16 // BOP
STRIPE = NPAD // 16
CW = 128
NCH = NEXP
BN = 256
NBLK = NPAD // BN

_F32 = jnp.float32
_BF16 = jnp.bfloat16
_I32 = jnp.int32


def _mesh():
    return plsc.VectorSubcoreMesh(
        core_axis_name="c", subcore_axis_name="s", num_cores=2, num_subcores=16
    )


def _sc_prep_body(x_hbm, nmap_hbm, dst_hbm, xs_out, degp_out,
                  nidx_v, didx_v, ones_v, rows_v, zb_v, dacc_sh, sem):
    c = lax.axis_index("c")
    s = lax.axis_index("s")
    wid = s * 2 + c
    lane = lax.iota(_I32, 16)
    e0 = jnp.where(lane == 0, 1.0, 0.0).astype(_F32)
    zv = jnp.zeros((16,), _F32)

    @pl.loop(0, B)
    def _init_ones(i):
        ones_v[i, :] = e0

    @pl.loop(0, 128)
    def _init_zb(i):
        zb_v[i, :] = zv

    for z in range(5):
        pltpu.sync_copy(zb_v, dacc_sh.at[pl.ds(s * STRIPE + z * 128, 128)])
    plsc.subcore_barrier()

    dbase = wid * (E_PAD // 32)

    @pl.loop(0, NB_DEG)
    def _deg(b):
        off = pl.multiple_of(dbase + b * B, B)
        pltpu.sync_copy(dst_hbm.at[pl.ds(off, B)], didx_v)
        pltpu.sync_copy(ones_v, dacc_sh.at[didx_v], add=True)

    nbase = wid * (NPAD // 32)
    for q in range(5):
        qoff = pl.multiple_of(nbase + q * 64, 8)
        pltpu.sync_copy(nmap_hbm.at[pl.ds(qoff, 64)], nidx_v)
        pltpu.async_copy(x_hbm.at[nidx_v], rows_v, sem).wait()
        pltpu.sync_copy(rows_v, xs_out.at[pl.ds(qoff, 64)])

    plsc.subcore_barrier()
    pltpu.sync_copy(
        dacc_sh.at[pl.ds(s * STRIPE, STRIPE)],
        degp_out.at[c].at[pl.ds(s * STRIPE, STRIPE)],
    )


def _sc_prep(x, nmap_pad, dst_pad):
    f = pl.kernel(
        _sc_prep_body,
        out_type=(
            jax.ShapeDtypeStruct((NPAD, D), _F32),
            jax.ShapeDtypeStruct((2, NPAD, 16), _F32),
        ),
        mesh=_mesh(),
        scratch_types=[
            pltpu.VMEM((64,), _I32),
            pltpu.VMEM((B,), _I32),
            pltpu.VMEM((B, 16), _F32),
            pltpu.VMEM((64, D), _F32),
            pltpu.VMEM((128, 16), _F32),
            pltpu.VMEM_SHARED((NPAD, 16), _F32),
            pltpu.SemaphoreType.DMA,
        ],
        compiler_params=pltpu.CompilerParams(use_tc_tiling_on_sc=False),
    )
    return f(x, nmap_pad, dst_pad)


def _sc_agg_body(h_hbm, src_hbm3, dst_hbm3, dummy_hbm, s_out,
                 srcv, dstv, rows0, rows1, rows2, rows3, rows4, zb_v, acc_sh,
                 gsem0, gsem1, gsem2, gsem3, gsem4,
                 ssem0, ssem1, ssem2, ssem3, ssem4):
    c = lax.axis_index("c")
    s = lax.axis_index("s")
    zv = jnp.zeros((32,), _BF16)
    rows = (rows0, rows1, rows2, rows3, rows4)
    gsem = (gsem0, gsem1, gsem2, gsem3, gsem4)
    ssem = (ssem0, ssem1, ssem2, ssem3, ssem4)

    @pl.loop(0, 64)
    def _init_zb(i):
        for j in range(CW // 32):
            zb_v[i, pl.ds(j * 32, 32)] = zv

    pltpu.sync_copy(src_hbm3.at[s], srcv)
    pltpu.sync_copy(dst_hbm3.at[s], dstv)

    for q in range(NCH):
        def _fire_g(st, b, _q=q):
            pltpu.async_copy(h_hbm.at[_q].at[srcv.at[b]], rows[st], gsem[st])

        def _fire_s(st, b):
            pltpu.async_copy(rows[st], acc_sh.at[dstv.at[b]], ssem[st],
                             add=True)

        def _drain_g(st):
            pltpu.make_async_copy(dummy_hbm, rows[st], gsem[st]).wait()

        def _drain_s(st):
            pltpu.make_async_copy(rows[st], acc_sh.at[pl.ds(0, BOP)],
                                  ssem[st]).wait()

        @pl.when(c == q // 4)
        def _chunk():
            for z in range(10):
                pltpu.sync_copy(zb_v, acc_sh.at[pl.ds(s * STRIPE + z * 64, 64)])
            plsc.subcore_barrier()

            for st in range(5):
                _fire_g(st, st)

            @pl.loop(0, (NBOP - 5) // 5)
            def _quints(t):
                b = 5 * t
                for st in range(5):
                    _drain_g(st)
                    _fire_s(st, b + st)
                for st in range(5):
                    _drain_s(st)
                    _fire_g(st, b + 5 + st)

            for st in range(5):
                _drain_g(st)
                _fire_s(st, NBOP - 5 + st)
            for st in range(5):
                _drain_s(st)

            plsc.subcore_barrier()
            pltpu.sync_copy(
                acc_sh.at[pl.ds(s * STRIPE, STRIPE)],
                s_out.at[q].at[pl.ds(s * STRIPE, STRIPE)],
            )


_SC_AGG_SINGLETON = None


def _sc_agg(h, src3, dst3):
    global _SC_AGG_SINGLETON
    if _SC_AGG_SINGLETON is not None:
        return _SC_AGG_SINGLETON(h, src3, dst3, jnp.zeros((BOP, CW), _BF16))
    f = pl.kernel(
        _sc_agg_body,
        out_type=jax.ShapeDtypeStruct((NCH, NPAD, CW), _BF16),
        mesh=_mesh(),
        scratch_types=[
            pltpu.VMEM((NBOP, BOP), _I32),
            pltpu.VMEM((NBOP, BOP), _I32),
            pltpu.VMEM((BOP, CW), _BF16),
            pltpu.VMEM((BOP, CW), _BF16),
            pltpu.VMEM((BOP, CW), _BF16),
            pltpu.VMEM((BOP, CW), _BF16),
            pltpu.VMEM((BOP, CW), _BF16),
            pltpu.VMEM((64, CW), _BF16),
            pltpu.VMEM_SHARED((NPAD, CW), _BF16),
            pltpu.SemaphoreType.DMA,
            pltpu.SemaphoreType.DMA,
            pltpu.SemaphoreType.DMA,
            pltpu.SemaphoreType.DMA,
            pltpu.SemaphoreType.DMA,
            pltpu.SemaphoreType.DMA,
            pltpu.SemaphoreType.DMA,
            pltpu.SemaphoreType.DMA,
            pltpu.SemaphoreType.DMA,
            pltpu.SemaphoreType.DMA,
        ],
        compiler_params=pltpu.CompilerParams(use_tc_tiling_on_sc=False),
    )
    _SC_AGG_SINGLETON = f
    return f(h, src3, dst3, jnp.zeros((BOP, CW), _BF16))


def _tc1_body(xs_ref, dga_ref, dgb_ref, sg_ref, w1_ref, h1_ref, g_ref):
    xs = xs_ref[...]
    logits = lax.dot_general(xs, sg_ref[...], (((1,), (1,)), ((), ())),
                             preferred_element_type=_F32)
    m = jnp.max(logits, axis=1, keepdims=True)
    p = jnp.exp(logits - m)
    g = p / jnp.sum(p, axis=1, keepdims=True)
    g_ref[...] = g
    dis = lax.rsqrt(dga_ref[...] + dgb_ref[...] + 1.0)
    for e in range(NEXP):
        he = lax.dot_general(g[:, e:e + 1] * xs, w1_ref[e],
                             (((1,), (0,)), ((), ())),
                             preferred_element_type=_F32)
        h1_ref[e] = (dis * he).astype(_BF16)


def _tc1(xs, dga, dgb, softgate, w1):
    return pl.pallas_call(
        _tc1_body,
        grid=(NBLK,),
        in_specs=[
            pl.BlockSpec((BN, D), lambda i: (i, 0)),
            pl.BlockSpec((BN, 1), lambda i: (i, 0)),
            pl.BlockSpec((BN, 1), lambda i: (i, 0)),
            pl.BlockSpec((NEXP, D), lambda i: (0, 0)),
            pl.BlockSpec((NEXP, D, D), lambda i: (0, 0, 0)),
        ],
        out_specs=[
            pl.BlockSpec((NCH, BN, CW), lambda i: (0, i, 0)),
            pl.BlockSpec((BN, NEXP), lambda i: (i, 0)),
        ],
        out_shape=[
            jax.ShapeDtypeStruct((NCH, NPAD, CW), _BF16),
            jax.ShapeDtypeStruct((NPAD, NEXP), _F32),
        ],
    )(xs, dga, dgb, softgate, w1)


def _tc2_body(s1_ref, h1_ref, dga_ref, dgb_ref, b1_ref, w2_ref, h2_ref):
    dis = lax.rsqrt(dga_ref[...] + dgb_ref[...] + 1.0)
    for e in range(NEXP):
        u = s1_ref[e].astype(_F32) + h1_ref[e].astype(_F32)
        z = jnp.maximum(dis * u + b1_ref[e], 0.0)
        w = dis * lax.dot_general(z, w2_ref[e], (((1,), (0,)), ((), ())),
                                  preferred_element_type=_F32)
        h2_ref[e] = w.astype(_BF16)


def _tc2(s1, h1, dga, dgb, b1, w2):
    return pl.pallas_call(
        _tc2_body,
        grid=(NBLK,),
        in_specs=[
            pl.BlockSpec((NCH, BN, CW), lambda i: (0, i, 0)),
            pl.BlockSpec((NCH, BN, CW), lambda i: (0, i, 0)),
            pl.BlockSpec((BN, 1), lambda i: (i, 0)),
            pl.BlockSpec((BN, 1), lambda i: (i, 0)),
            pl.BlockSpec((NEXP, D), lambda i: (0, 0)),
            pl.BlockSpec((NEXP, D, D), lambda i: (0, 0, 0)),
        ],
        out_specs=pl.BlockSpec((NCH, BN, CW), lambda i: (0, i, 0)),
        out_shape=jax.ShapeDtypeStruct((NCH, NPAD, CW), _BF16),
    )(s1, h1, dga, dgb, b1, w2)


def _tc3_body(s2_ref, h2_ref, dga_ref, dgb_ref, b2_ref, g_ref, out_ref):
    dis = lax.rsqrt(dga_ref[...] + dgb_ref[...] + 1.0)
    g = g_ref[...]
    acc = jnp.zeros((BN, D), _F32)
    for e in range(NEXP):
        u = s2_ref[e].astype(_F32) + h2_ref[e].astype(_F32)
        acc = acc + g[:, e:e + 1] * (dis * u + b2_ref[e])
    out_ref[...] = acc


def _tc3(s2, h2, dga, dgb, b2, g):
    return pl.pallas_call(
        _tc3_body,
        grid=(NBLK,),
        in_specs=[
            pl.BlockSpec((NCH, BN, CW), lambda i: (0, i, 0)),
            pl.BlockSpec((NCH, BN, CW), lambda i: (0, i, 0)),
            pl.BlockSpec((BN, 1), lambda i: (i, 0)),
            pl.BlockSpec((BN, 1), lambda i: (i, 0)),
            pl.BlockSpec((NEXP, D), lambda i: (0, 0)),
            pl.BlockSpec((BN, NEXP), lambda i: (i, 0)),
        ],
        out_specs=pl.BlockSpec((BN, D), lambda i: (i, 0)),
        out_shape=jax.ShapeDtypeStruct((NPAD, D), _F32),
    )(s2, h2, dga, dgb, b2, g)


def kernel(x, node_map, edge_index, softgate, W1, b1, W2, b2):
    nm = node_map.astype(_I32)
    src = edge_index[0].astype(_I32)
    dst = edge_index[1].astype(_I32)
    nm_pad = jnp.concatenate([nm, jnp.zeros((NPAD - N,), _I32)])
    src_pad = jnp.concatenate([src, jnp.zeros((E_PAD - E,), _I32)])
    dst_pad = jnp.concatenate([dst, jnp.full((E_PAD - E,), TRASH, _I32)])
    src3 = src_pad.reshape(16, NBOP, BOP)
    dst3 = dst_pad.reshape(16, NBOP, BOP)

    xs_pad, degp = _sc_prep(x, nm_pad, dst_pad)
    dga = degp[0, :, 0:1]
    dgb = degp[1, :, 0:1]

    h1, g = _tc1(xs_pad, dga, dgb, softgate, W1)
    s1 = _sc_agg(h1, src3, dst3)
    h2 = _tc2(s1, h1, dga, dgb, b1, W2)
    s2 = _sc_agg(h2, src3, dst3)
    out_full = _tc3(s2, h2, dga, dgb, b2, g)
    return out_full[:N]
